# baseline (speedup 1.0000x reference)
import functools

import jax
import jax.numpy as jnp
from jax import lax
from jax.experimental import pallas as pl
from jax.experimental.pallas import tpu as pltpu
from jax.experimental.pallas import tpu_sc as plsc

HIDDEN = 64
MARGIN = 1.0
BATCH = 16384

L = 16
NC = 2
NS = 16
NW = NC * NS
BPW = BATCH // NW
CHUNK = 128
NCHUNK = BPW // CHUNK


def _score_chunk(srow, rrow, orow, score_v, score_off):
    for g in range(CHUNK // L):
        rowid = lax.iota(jnp.int32, L) + g * L

        def dbody(j, acc):
            for u in range(4):
                col = jnp.full((L,), 0, jnp.int32) + (j * 4 + u)
                s = plsc.load_gather(srow, [rowid, col])
                r = plsc.load_gather(rrow, [rowid, col])
                o = plsc.load_gather(orow, [rowid, col])
                acc = acc + jnp.abs(s + r - o)
            return acc

        acc = lax.fori_loop(0, HIDDEN // 4, dbody, jnp.zeros((L,), jnp.float32))
        score_v[pl.ds(score_off + g * L, L)] = acc


def _tec_body(idx_hbm, ent_hbm, rel_hbm, out_hbm,
              idx_v, srow, rrow, orow, pos_v, neg_v, out_v, sem):
    wid = lax.axis_index("s") * NC + lax.axis_index("c")
    base = wid * BPW

    pltpu.sync_copy(idx_hbm.at[wid], idx_v)

    for side, score_v in ((0, pos_v), (1, neg_v)):
        for c in range(NCHUNK):
            cp_s = pltpu.async_copy(ent_hbm.at[idx_v.at[3 * side + 0, c]], srow, sem)
            cp_r = pltpu.async_copy(rel_hbm.at[idx_v.at[3 * side + 1, c]], rrow, sem)
            cp_o = pltpu.async_copy(ent_hbm.at[idx_v.at[3 * side + 2, c]], orow, sem)
            cp_s.wait()
            cp_r.wait()
            cp_o.wait()
            _score_chunk(srow, rrow, orow, score_v, c * CHUNK)

    def loss_body(k, _):
        p = pos_v[pl.ds(k * L, L)]
        n = neg_v[pl.ds(k * L, L)]
        out_v[pl.ds(k * L, L)] = jnp.maximum(p - n + MARGIN, 0.0)
        return 0

    lax.fori_loop(0, BPW // L, loss_body, 0)
    pltpu.sync_copy(out_v, out_hbm.at[pl.ds(base, BPW)])


@jax.jit
def _transe_sc(idx, entity_embedding, relation_embedding):
    run = pl.kernel(
        _tec_body,
        out_type=jax.ShapeDtypeStruct((BATCH,), jnp.float32),
        mesh=plsc.VectorSubcoreMesh(core_axis_name="c", subcore_axis_name="s"),
        compiler_params=pltpu.CompilerParams(
            needs_layout_passes=False, use_tc_tiling_on_sc=False),
        scratch_types=[
            pltpu.VMEM((6, NCHUNK, CHUNK), jnp.int32),
            pltpu.VMEM((CHUNK, HIDDEN), jnp.float32),
            pltpu.VMEM((CHUNK, HIDDEN), jnp.float32),
            pltpu.VMEM((CHUNK, HIDDEN), jnp.float32),
            pltpu.VMEM((BPW,), jnp.float32),
            pltpu.VMEM((BPW,), jnp.float32),
            pltpu.VMEM((BPW,), jnp.float32),
            pltpu.SemaphoreType.DMA,
        ],
    )
    return run(idx, entity_embedding, relation_embedding)


def kernel(positive_triple, negative_triple, entity_embedding, relation_embedding):
    pos = positive_triple.astype(jnp.int32)
    neg = negative_triple.astype(jnp.int32)
    idx6 = jnp.stack(
        [pos[:, 0], pos[:, 1], pos[:, 2], neg[:, 0], neg[:, 1], neg[:, 2]], axis=0)
    idx = jnp.transpose(idx6.reshape(6, NW, NCHUNK, CHUNK), (1, 0, 2, 3))
    return _transe_sc(idx, entity_embedding, relation_embedding)


# speedup vs baseline: 1.2799x; 1.2799x over previous
import jax
import jax.numpy as jnp
from jax import lax
from jax.experimental import pallas as pl
from jax.experimental.pallas import tpu as pltpu
from jax.experimental.pallas import tpu_sc as plsc

HIDDEN = 64
MARGIN = 1.0
BATCH = 16384
NROW = 1000000

L = 16
NC = 2
NS = 16
NW = NC * NS
BPW = BATCH // NW
CHUNK = 32
NCHUNK = BPW // CHUNK
IPW = 6 * BPW
LEAD = 16


def _score_chunk(idx_v, off, sbuf, rbuf, obuf, score_v, score_off):
    for g in range(CHUNK // L):
        rowid = lax.iota(jnp.int32, L) + g * L
        ssub = idx_v[pl.ds(off + 0 * CHUNK + g * L, L)] & 7
        rsub = idx_v[pl.ds(off + 1 * CHUNK + g * L, L)] & 7
        osub = idx_v[pl.ds(off + 2 * CHUNK + g * L, L)] & 7

        def dbody(j, acc):
            for u in range(4):
                d = jnp.full((L,), j * 4 + u, jnp.int32)
                s = plsc.load_gather(sbuf, [rowid, ssub, d])
                r = plsc.load_gather(rbuf, [rowid, rsub, d])
                o = plsc.load_gather(obuf, [rowid, osub, d])
                acc = acc + jnp.abs(s + r - o)
            return acc

        acc = lax.fori_loop(0, HIDDEN // 4, dbody, jnp.zeros((L,), jnp.float32))
        score_v[pl.ds(score_off + g * L, L)] = acc


def _tec_body(idx_hbm, ent_hbm, rel_hbm, out_hbm,
              idx_v, sbuf, rbuf, obuf, score_v, out_v, sem):
    wid = lax.axis_index("s") * NC + lax.axis_index("c")
    base = wid * BPW

    ent2 = ent_hbm.at[pl.ds(0, NROW)].reshape(NROW // 8, 8, HIDDEN)
    rel2 = rel_hbm.at[pl.ds(0, NROW)].reshape(NROW // 8, 8, HIDDEN)

    pltpu.sync_copy(idx_hbm.at[pl.ds(wid * IPW, IPW)], idx_v)

    def chunk_body(p, _):
        off = p * (3 * CHUNK)

        def issue(g, _):
            sv = lax.shift_right_logical(
                idx_v[pl.ds(off + 0 * CHUNK + g * L, L)], 3)
            rv = lax.shift_right_logical(
                idx_v[pl.ds(off + 1 * CHUNK + g * L, L)], 3)
            ov = lax.shift_right_logical(
                idx_v[pl.ds(off + 2 * CHUNK + g * L, L)], 3)
            for j in range(L):
                i = g * L + j
                pltpu.async_copy(ent2.at[sv[j]], sbuf.at[i], sem)
                pltpu.async_copy(rel2.at[rv[j]], rbuf.at[i], sem)
                pltpu.async_copy(ent2.at[ov[j]], obuf.at[i], sem)

            @pl.when(g > 0)
            def _():
                for _k in range(3 * L):
                    pltpu.make_async_copy(ent2.at[0], sbuf.at[0], sem).wait()

            return 0

        lax.fori_loop(0, CHUNK // L, issue, 0)

        for _k in range(3 * L):
            pltpu.make_async_copy(ent2.at[0], sbuf.at[0], sem).wait()

        _score_chunk(idx_v, off, sbuf, rbuf, obuf, score_v, p * CHUNK)
        return 0

    lax.fori_loop(0, 2 * NCHUNK, chunk_body, 0)

    def loss_body(k, _):
        p = score_v[pl.ds(k * L, L)]
        n = score_v[pl.ds(BPW + k * L, L)]
        out_v[pl.ds(k * L, L)] = jnp.maximum(p - n + MARGIN, 0.0)
        return 0

    lax.fori_loop(0, BPW // L, loss_body, 0)
    pltpu.sync_copy(out_v, out_hbm.at[pl.ds(base, BPW)])


@jax.jit
def _transe_sc(idx, entity_embedding, relation_embedding):
    run = pl.kernel(
        _tec_body,
        out_type=jax.ShapeDtypeStruct((BATCH,), jnp.float32),
        mesh=plsc.VectorSubcoreMesh(core_axis_name="c", subcore_axis_name="s"),
        compiler_params=pltpu.CompilerParams(needs_layout_passes=False),
        scratch_types=[
            pltpu.VMEM((IPW,), jnp.int32),
            pltpu.VMEM((CHUNK, 8, HIDDEN), jnp.float32),
            pltpu.VMEM((CHUNK, 8, HIDDEN), jnp.float32),
            pltpu.VMEM((CHUNK, 8, HIDDEN), jnp.float32),
            pltpu.VMEM((2 * BPW,), jnp.float32),
            pltpu.VMEM((BPW,), jnp.float32),
            pltpu.SemaphoreType.DMA,
        ],
    )
    return run(idx, entity_embedding, relation_embedding)


def kernel(positive_triple, negative_triple, entity_embedding, relation_embedding):
    pos = positive_triple.astype(jnp.int32)
    neg = negative_triple.astype(jnp.int32)
    idx6 = jnp.stack(
        [pos[:, 0], pos[:, 1], pos[:, 2], neg[:, 0], neg[:, 1], neg[:, 2]], axis=0)
    idx = jnp.transpose(
        idx6.reshape(2, 3, NW, NCHUNK, CHUNK), (2, 0, 3, 1, 4)).reshape(-1)
    return _transe_sc(idx, entity_embedding, relation_embedding)
